# Initial kernel scaffold; baseline (speedup 1.0000x reference)
import jax
import jax.numpy as jnp
from jax.experimental import pallas as pl


def kernel(x, router_logits, w3_w1_weight, w2_weight):
    raise NotImplementedError("write your pallas kernel here")


# speedup vs baseline: 1.2237x; 1.2237x over previous
import functools

import jax
import jax.numpy as jnp
from jax import lax
from jax.experimental import pallas as pl
from jax.experimental.pallas import tpu as pltpu
from jax.experimental.pallas import tpu_sc as plsc

E = 8
K = 2
H = 1024
I = 2048
T = 2048
B = 256
G = 24
PAD = G * B


def _routing_body(logits_ref, pos1_ref, pos2_ref, stok_ref, sscale_ref,
                  be_ref, bv_ref):
    logits = logits_ref[...]
    m = jnp.max(logits, axis=-1, keepdims=True)
    ex = jnp.exp(logits - m)
    p = ex / jnp.sum(ex, axis=-1, keepdims=True)

    iota_e = lax.broadcasted_iota(jnp.int32, (T, E), 1)
    m1 = jnp.max(p, axis=-1, keepdims=True)
    i1 = jnp.min(jnp.where(p == m1, iota_e, E), axis=-1, keepdims=True)
    oh1 = (iota_e == i1).astype(jnp.float32)
    p2m = jnp.where(iota_e == i1, -jnp.inf, p)
    m2 = jnp.max(p2m, axis=-1, keepdims=True)
    i2 = jnp.min(jnp.where(p2m == m2, iota_e, E), axis=-1, keepdims=True)
    oh2 = (iota_e == i2).astype(jnp.float32)
    s1 = jnp.sum(p * oh1, axis=-1)
    s2 = jnp.sum(p2m * oh2, axis=-1)

    r = lax.broadcasted_iota(jnp.int32, (T, T), 0)
    c = lax.broadcasted_iota(jnp.int32, (T, T), 1)
    tri = (c < r).astype(jnp.float32)
    csum1 = jnp.dot(tri, oh1, preferred_element_type=jnp.float32)
    csum2 = jnp.dot(tri, oh2, preferred_element_type=jnp.float32)
    tot1 = jnp.sum(oh1, axis=0)
    tot2 = jnp.sum(oh2, axis=0)
    counts = tot1 + tot2

    nb = jnp.floor((counts + (B - 1)) / B)
    re8 = lax.broadcasted_iota(jnp.int32, (E, E), 0)
    ce8 = lax.broadcasted_iota(jnp.int32, (E, E), 1)
    tri8 = (ce8 < re8).astype(jnp.float32)
    startblk = jnp.dot(tri8, nb, preferred_element_type=jnp.float32)
    start_rows = startblk * B

    rank1 = jnp.sum(csum1 * oh1, axis=-1)
    rank2 = jnp.sum(oh2 * tot1[None, :], axis=-1) + jnp.sum(csum2 * oh2, axis=-1)
    pos1 = jnp.sum(oh1 * start_rows[None, :], axis=-1) + rank1
    pos2 = jnp.sum(oh2 * start_rows[None, :], axis=-1) + rank2
    pos1_ref[...] = pos1.astype(jnp.int32)
    pos2_ref[...] = pos2.astype(jnp.int32)

    total_active = jnp.sum(nb)
    gmat = lax.broadcasted_iota(jnp.int32, (G, E), 0).astype(jnp.float32)
    be = jnp.sum((gmat >= startblk[None, :]).astype(jnp.float32), axis=-1) - 1.0
    gvec = lax.broadcasted_iota(jnp.int32, (G, 1), 0).astype(jnp.float32)[:, 0]
    valid = (gvec < total_active)
    evec = lax.broadcasted_iota(jnp.int32, (E, 1), 0).astype(jnp.float32)[:, 0]
    last_e = jnp.max(jnp.where(nb > 0, evec, 0.0))
    be = jnp.where(valid, be, last_e)
    be_ref[...] = be.astype(jnp.int32)
    bv_ref[...] = valid.astype(jnp.int32)

    tvec = lax.broadcasted_iota(jnp.int32, (T, 1), 0).astype(jnp.float32)[:, 0]
    cols1 = jnp.stack([tvec, s1], axis=1)
    cols2 = jnp.stack([tvec, s2], axis=1)
    SB = 1024
    for sb in range(PAD // SB):
        slot = lax.broadcasted_iota(jnp.int32, (SB, T), 0).astype(jnp.float32) + (
            sb * SB)
        m1b = (slot == pos1[None, :]).astype(jnp.float32)
        m2b = (slot == pos2[None, :]).astype(jnp.float32)
        a = (jnp.dot(m1b, cols1, preferred_element_type=jnp.float32)
             + jnp.dot(m2b, cols2, preferred_element_type=jnp.float32))
        stok_ref[pl.ds(sb * SB, SB)] = a[:, 0].astype(jnp.int32)
        sscale_ref[pl.ds(sb * SB, SB)] = a[:, 1]


_routing = pl.pallas_call(
    _routing_body,
    out_shape=(
        jax.ShapeDtypeStruct((T,), jnp.int32),
        jax.ShapeDtypeStruct((T,), jnp.int32),
        jax.ShapeDtypeStruct((PAD,), jnp.int32),
        jax.ShapeDtypeStruct((PAD,), jnp.float32),
        jax.ShapeDtypeStruct((G,), jnp.int32),
        jax.ShapeDtypeStruct((G,), jnp.int32),
    ),
)


_NC, _NS = 2, 16
_NW = _NC * _NS
_ROWS_W = PAD // _NW
_CH = 48
_TOK_W = T // _NW
_TCH = 32

@functools.cache
def _sc_kernels():
    mesh = plsc.VectorSubcoreMesh(
        core_axis_name="c", subcore_axis_name="s", num_cores=_NC)

    @functools.partial(
        pl.kernel, mesh=mesh,
        out_type=jax.ShapeDtypeStruct((PAD, H), jnp.float32),
        scratch_types=[
            pltpu.VMEM((_ROWS_W,), jnp.int32),
            pltpu.VMEM((_CH, H), jnp.float32),
            pltpu.VMEM((_CH, H), jnp.float32),
            pltpu.SemaphoreType.DMA,
            pltpu.SemaphoreType.DMA,
        ],
    )
    def dispatch(x_hbm, stok_hbm, xs_hbm, idx_v, buf0, buf1, sem0, sem1):
        wid = lax.axis_index("s") * _NC + lax.axis_index("c")
        base = wid * _ROWS_W
        pltpu.sync_copy(stok_hbm.at[pl.ds(base, _ROWS_W)], idx_v)
        bufs = (buf0, buf1)
        sems = (sem0, sem1)
        nch = _ROWS_W // _CH
        cps = [None] * nch
        for ch in range(nch):
            cps[ch] = pltpu.async_copy(
                x_hbm.at[idx_v.at[pl.ds(ch * _CH, _CH)]],
                bufs[ch % 2], sems[ch % 2])
            if ch >= 1:
                cps[ch - 1].wait()
                pltpu.sync_copy(bufs[(ch - 1) % 2],
                                xs_hbm.at[pl.ds(base + (ch - 1) * _CH, _CH)])
        cps[nch - 1].wait()
        pltpu.sync_copy(bufs[(nch - 1) % 2],
                        xs_hbm.at[pl.ds(base + (nch - 1) * _CH, _CH)])

    @functools.partial(
        pl.kernel, mesh=mesh,
        out_type=jax.ShapeDtypeStruct((T, H), jnp.float32),
        scratch_types=[
            pltpu.VMEM((_TOK_W,), jnp.int32),
            pltpu.VMEM((_TOK_W,), jnp.int32),
            pltpu.VMEM((_TCH, H), jnp.float32),
            pltpu.VMEM((_TCH, H), jnp.float32),
            pltpu.SemaphoreType.DMA,
            pltpu.SemaphoreType.DMA,
        ],
    )
    def combine(ys_hbm, p1_hbm, p2_hbm, out_hbm, i1_v, i2_v, bufa, bufb,
                sema, semb):
        wid = lax.axis_index("s") * _NC + lax.axis_index("c")
        base = wid * _TOK_W
        pltpu.sync_copy(p1_hbm.at[pl.ds(base, _TOK_W)], i1_v)
        pltpu.sync_copy(p2_hbm.at[pl.ds(base, _TOK_W)], i2_v)
        for ch in range(_TOK_W // _TCH):
            cpa = pltpu.async_copy(
                ys_hbm.at[i1_v.at[pl.ds(ch * _TCH, _TCH)]], bufa, sema)
            cpb = pltpu.async_copy(
                ys_hbm.at[i2_v.at[pl.ds(ch * _TCH, _TCH)]], bufb, semb)
            cpa.wait()
            cpb.wait()

            def add_row(r, _):
                def add_lane(c2, _):
                    bufa[r, pl.ds(c2 * 16, 16)] = (
                        bufa[r, pl.ds(c2 * 16, 16)]
                        + bufb[r, pl.ds(c2 * 16, 16)])
                    return 0
                return lax.fori_loop(0, H // 16, add_lane, 0, unroll=4)

            lax.fori_loop(0, _TCH, add_row, 0)
            pltpu.sync_copy(bufa, out_hbm.at[pl.ds(base + ch * _TCH, _TCH)])

    return dispatch, combine


def _ffn_body(be_ref, bv_ref, xs_ref, w31_ref, w2_ref, sc_ref, ys_ref):
    g = pl.program_id(0)

    @pl.when(bv_ref[g] != 0)
    def _():
        xb = xs_ref[...]
        w31 = w31_ref[0]
        h = lax.dot_general(xb, w31, (((1,), (1,)), ((), ())),
                            preferred_element_type=jnp.float32)
        h3 = h[:, :I]
        h1 = h[:, I:]
        act = h1 * jax.nn.sigmoid(h1) * h3
        w2 = w2_ref[0]
        o = lax.dot_general(act, w2, (((1,), (1,)), ((), ())),
                            preferred_element_type=jnp.float32)
        ys_ref[...] = o * sc_ref[0, 0][:, None]


def _ffn(be, bv, xs, w31, w2, sscale):
    grid_spec = pltpu.PrefetchScalarGridSpec(
        num_scalar_prefetch=2,
        grid=(G,),
        in_specs=[
            pl.BlockSpec((B, H), lambda g, be, bv: (g, 0)),
            pl.BlockSpec((1, 2 * I, H), lambda g, be, bv: (be[g], 0, 0)),
            pl.BlockSpec((1, H, I), lambda g, be, bv: (be[g], 0, 0)),
            pl.BlockSpec((1, 1, B), lambda g, be, bv: (g, 0, 0)),
        ],
        out_specs=pl.BlockSpec((B, H), lambda g, be, bv: (g, 0)),
    )
    return pl.pallas_call(
        _ffn_body,
        grid_spec=grid_spec,
        out_shape=jax.ShapeDtypeStruct((PAD, H), jnp.float32),
    )(be, bv, xs, w31, w2, sscale.reshape(G, 1, B))


def kernel(x, router_logits, w3_w1_weight, w2_weight):
    dispatch, combine = _sc_kernels()
    pos1, pos2, stok, sscale, be, bv = _routing(router_logits)
    xs = dispatch(x, stok)
    ys = _ffn(be, bv, xs, w3_w1_weight, w2_weight, sscale)
    return combine(ys, pos1, pos2)
